# baseline (speedup 1.0000x reference)
import functools

import jax
import jax.numpy as jnp
from jax import lax
from jax.experimental import pallas as pl
from jax.experimental.pallas import tpu as pltpu
from jax.experimental.pallas import tpu_sc as plsc

_N = 50000
_E = 800000
_G = 128
_D = 32
_XD = 78
_SEQ = 1000
_KSZ = 8
_NF = 32
_CONV_T = _SEQ - _KSZ + 1

_NC, _NS = 2, 16
_NW = _NC * _NS
_EPW = _E // _NW
_EP = 819200
_IDXW = 128
_NROWS = _EP // _IDXW
_RPW = _NROWS // _NW
_BPW = _RPW // 8
_NPAD = 50048
_ZPW = _NPAD // _NS
_R = 5

_BN = 1.0 / (1.0 + 1e-5) ** 0.5

_f32 = jnp.float32


def _edge_body(u_hbm, src_hbm, dst_hbm, zero_hbm, out_hbm,
               src_v, dst_v, rows_v, acc_sh, isem, gsem, ssem):
    c = lax.axis_index("c")
    s = lax.axis_index("s")
    w = s * _NC + c
    base = w * _RPW

    def idx_load(blk, buf):
        pltpu.async_copy(src_hbm.at[pl.ds(base + blk * 8, 8)],
                         src_v.at[buf], isem)
        pltpu.async_copy(dst_hbm.at[pl.ds(base + blk * 8, 8)],
                         dst_v.at[buf], isem)

    def idx_wait():
        pltpu.make_async_copy(src_hbm.at[pl.ds(0, 8)], src_v.at[0],
                              isem).wait()
        pltpu.make_async_copy(dst_hbm.at[pl.ds(0, 8)], dst_v.at[0],
                              isem).wait()

    def g_issue(buf, row, slot):
        pltpu.async_copy(u_hbm.at[src_v.at[buf, row]], rows_v.at[slot],
                         gsem)

    def g_wait():
        pltpu.make_async_copy(u_hbm.at[src_v.at[0, 0]], rows_v.at[0],
                              gsem).wait()

    def s_issue(buf, row, slot):
        pltpu.sync_copy(rows_v.at[slot], acc_sh.at[dst_v.at[buf, row]],
                        add=True)

    with jax.named_scope("zero_phase"):
        pltpu.sync_copy(zero_hbm.at[pl.ds(s * _ZPW, _ZPW)],
                        acc_sh.at[pl.ds(s * _ZPW, _ZPW)])
        plsc.subcore_barrier()

    def item(B, r, *, first_block=False, last_block=False):
        bb = lax.rem(B, 2)
        nb = lax.rem(B + 1, 2)
        j = B * 8 + r
        if r == 1 and not last_block:
            idx_load(B + 1, nb)
        if r == 4 and not last_block:
            idx_wait()
        if not (last_block and r > 4):
            if r <= 4:
                g_issue(bb, r + 3, lax.rem(j + 3, _R))
            else:
                g_issue(nb, r - 5, lax.rem(j + 3, _R))
        g_wait()
        s_issue(bb, r, lax.rem(j, _R))

    with jax.named_scope("edge_loop"):
        zero = jnp.zeros((), jnp.int32)
        idx_load(0, 0)
        idx_wait()
        for m in range(3):
            g_issue(0, m, m)
        for r in range(8):
            item(zero, r, first_block=True)

        def blk(B, carry):
            for r in range(8):
                item(B, r)
            return carry

        lax.fori_loop(1, _BPW - 1, blk, 0)

        last = jnp.full((), _BPW - 1, jnp.int32)
        for r in range(8):
            item(last, r, last_block=True)
    with jax.named_scope("export_phase"):
        plsc.subcore_barrier()
        pltpu.sync_copy(acc_sh.at[pl.ds(s * _ZPW, _ZPW)],
                        out_hbm.at[c].at[pl.ds(s * _ZPW, _ZPW)])


@functools.cache
def _make_edge_call():
    return pl.kernel(
        _edge_body,
        out_type=jax.ShapeDtypeStruct((_NC, _NPAD, _D), _f32),
        mesh=plsc.VectorSubcoreMesh(core_axis_name="c", subcore_axis_name="s",
                                    num_cores=_NC, num_subcores=_NS),
        scratch_types=[
            pltpu.VMEM((2, 8, _IDXW), jnp.int32),
            pltpu.VMEM((2, 8, _IDXW), jnp.int32),
            pltpu.VMEM((_R, _IDXW, _D), _f32),
            pltpu.VMEM_SHARED((_NPAD, _D), _f32),
            pltpu.SemaphoreType.DMA,
            pltpu.SemaphoreType.DMA,
            pltpu.SemaphoreType.DMA,
        ],
        compiler_params=pltpu.CompilerParams(use_tc_tiling_on_sc=False),
    )


_NP4 = _N // 4
_PPAD = _NPAD // 4
_RB = 3128
_NRB = _PPAD // _RB


def _bd4(w):
    z = jnp.zeros(w.shape, w.dtype)
    rows = [jnp.concatenate([z] * k + [w] + [z] * (3 - k), axis=1)
            for k in range(4)]
    return jnp.concatenate(rows, axis=0)


def _t4(v):
    return jnp.concatenate([v, v, v, v], axis=1)


def _u0_body(x_ref, w_ref, o_ref):
    u = jnp.dot(x_ref[...], _bd4(w_ref[...]), preferred_element_type=_f32)
    o_ref[...] = jnp.concatenate(
        [u, jnp.zeros((_PPAD - _NP4, 128), _f32)], axis=0)


_u0_call = pl.pallas_call(
    _u0_body,
    in_specs=[
        pl.BlockSpec((_NP4, 4 * _XD), lambda: (0, 0)),
        pl.BlockSpec((_XD, _D), lambda: (0, 0)),
    ],
    out_specs=pl.BlockSpec((_PPAD, 128), lambda: (0, 0)),
    out_shape=jax.ShapeDtypeStruct((_PPAD, 128), _f32),
)


def _mlp(u_ref, p_ref, b1_ref, w2_ref, b2_ref, g_ref, be_ref):
    z = jnp.maximum(u_ref[...] + p_ref[0] + p_ref[1] + _t4(b1_ref[...]), 0.0)
    z = jnp.maximum(jnp.dot(z, _bd4(w2_ref[...]),
                            preferred_element_type=_f32)
                    + _t4(b2_ref[...]), 0.0)
    return z * (_t4(g_ref[...]) * _BN) + _t4(be_ref[...])


def _layer_body(u_ref, p_ref, b1_ref, w2_ref, b2_ref, g_ref, be_ref,
                w1n_ref, o_ref):
    h = _mlp(u_ref, p_ref, b1_ref, w2_ref, b2_ref, g_ref, be_ref)
    o_ref[...] = jnp.dot(h, _bd4(w1n_ref[...]),
                         preferred_element_type=_f32)


_layer_call = pl.pallas_call(
    _layer_body,
    grid=(_NRB,),
    in_specs=[
        pl.BlockSpec((_RB, 128), lambda i: (i, 0)),
        pl.BlockSpec((_NC, _RB, 128), lambda i: (0, i, 0)),
        pl.BlockSpec((1, _D), lambda i: (0, 0)),
        pl.BlockSpec((_D, _D), lambda i: (0, 0)),
        pl.BlockSpec((1, _D), lambda i: (0, 0)),
        pl.BlockSpec((1, _D), lambda i: (0, 0)),
        pl.BlockSpec((1, _D), lambda i: (0, 0)),
        pl.BlockSpec((_D, _D), lambda i: (0, 0)),
    ],
    out_specs=pl.BlockSpec((_RB, 128), lambda i: (i, 0)),
    out_shape=jax.ShapeDtypeStruct((_PPAD, 128), _f32),
)


def _layer4_body(u_ref, p_ref, b1_ref, w2_ref, b2_ref, g_ref, be_ref,
                 bt_ref, o_ref):
    h = _mlp(u_ref, p_ref, b1_ref, w2_ref, b2_ref, g_ref, be_ref)
    iota = lax.broadcasted_iota(jnp.int32, (1, _G), 1)
    part = jnp.zeros((_G, _D), _f32)
    for k in range(4):
        onek = (bt_ref[:, k:k + 1] == iota).astype(_f32)
        part = part + lax.dot_general(
            onek, h[:, 32 * k:32 * k + 32], (((0,), (0,)), ((), ())),
            preferred_element_type=_f32)
    i = pl.program_id(0)

    @pl.when(i == 0)
    def _init():
        o_ref[...] = part

    @pl.when(i > 0)
    def _acc():
        o_ref[...] += part


_layer4_call = pl.pallas_call(
    _layer4_body,
    grid=(_NRB,),
    in_specs=[
        pl.BlockSpec((_RB, 128), lambda i: (i, 0)),
        pl.BlockSpec((_NC, _RB, 128), lambda i: (0, i, 0)),
        pl.BlockSpec((1, _D), lambda i: (0, 0)),
        pl.BlockSpec((_D, _D), lambda i: (0, 0)),
        pl.BlockSpec((1, _D), lambda i: (0, 0)),
        pl.BlockSpec((1, _D), lambda i: (0, 0)),
        pl.BlockSpec((1, _D), lambda i: (0, 0)),
        pl.BlockSpec((_RB, 4), lambda i: (i, 0)),
    ],
    out_specs=pl.BlockSpec((_G, _D), lambda i: (0, 0)),
    out_shape=jax.ShapeDtypeStruct((_G, _D), _f32),
)


def _mt_body(a_ref, e_ref, o_ref):
    o_ref[...] = jnp.dot(a_ref[...], e_ref[...],
                         preferred_element_type=_f32)


_mt_call = pl.pallas_call(
    _mt_body,
    in_specs=[
        pl.BlockSpec((_KSZ * _NF, 128), lambda: (0, 0)),
        pl.BlockSpec((128, _D), lambda: (0, 0)),
    ],
    out_specs=pl.BlockSpec((_KSZ * _NF, _D), lambda: (0, 0)),
    out_shape=jax.ShapeDtypeStruct((_KSZ * _NF, _D), _f32),
)

_GB = 8


def _conv_body(t_ref, m_ref, cb_ref, o_ref):
    iota_v = lax.broadcasted_iota(jnp.int32, (_D, 1), 0)
    for g in range(_GB):
        tgt = t_ref[g]
        onehot = (tgt == iota_v).astype(_f32)
        acc = jnp.zeros((_NF, _CONV_T), _f32)
        for k in range(_KSZ):
            mt_k = m_ref[pl.ds(k * _NF, _NF), :]
            p = jnp.dot(mt_k, onehot, preferred_element_type=_f32)
            acc = acc + p[:, k:k + _CONV_T]
        o_ref[g] = jnp.maximum(acc + cb_ref[...], 0.0)


_conv_call = pl.pallas_call(
    _conv_body,
    grid=(_G // _GB,),
    in_specs=[
        pl.BlockSpec((_GB, 1, _SEQ), lambda i: (i, 0, 0)),
        pl.BlockSpec((_KSZ * _NF, _D), lambda i: (0, 0)),
        pl.BlockSpec((_NF, 1), lambda i: (0, 0)),
    ],
    out_specs=pl.BlockSpec((_GB, _NF, _CONV_T), lambda i: (i, 0, 0)),
    out_shape=jax.ShapeDtypeStruct((_G, _NF, _CONV_T), _f32),
)

_FLAT = _NF * _CONV_T
_LM = 1024
_H1 = 1024
_H2 = 256


def _head_body(pooled_ref, wxd_ref, bxd_ref, c_ref, wxt_ref, bxt_ref,
               drug_ref, prot_ref, w1a_ref, w1b_ref, w1c_ref, w1d_ref,
               b1_ref, w2_ref, b2_ref, w3_ref, b3_ref, o_ref):
    xd = jnp.maximum(jnp.dot(pooled_ref[...], wxd_ref[...],
                             preferred_element_type=_f32) + bxd_ref[...], 0.0)
    xt = jnp.maximum(jnp.dot(c_ref[...], wxt_ref[...],
                             preferred_element_type=_f32) + bxt_ref[...], 0.0)
    y = (jnp.dot(xd, w1a_ref[...], preferred_element_type=_f32)
         + jnp.dot(xt, w1b_ref[...], preferred_element_type=_f32)
         + jnp.dot(drug_ref[...], w1c_ref[...], preferred_element_type=_f32)
         + jnp.dot(prot_ref[...], w1d_ref[...], preferred_element_type=_f32)
         + b1_ref[...])
    y = jnp.maximum(y, 0.0)
    y = jnp.maximum(jnp.dot(y, w2_ref[...], preferred_element_type=_f32)
                    + b2_ref[...], 0.0)
    o_ref[...] = jnp.dot(y, w3_ref[...],
                         preferred_element_type=_f32) + b3_ref[...]


_head_call = pl.pallas_call(
    _head_body,
    in_specs=[
        pl.BlockSpec((_G, _D), lambda: (0, 0)),
        pl.BlockSpec((_D, 128), lambda: (0, 0)),
        pl.BlockSpec((1, 128), lambda: (0, 0)),
        pl.BlockSpec((_G, _FLAT), lambda: (0, 0)),
        pl.BlockSpec((_FLAT, 128), lambda: (0, 0)),
        pl.BlockSpec((1, 128), lambda: (0, 0)),
        pl.BlockSpec((_G, _LM), lambda: (0, 0)),
        pl.BlockSpec((_G, _LM), lambda: (0, 0)),
        pl.BlockSpec((128, _H1), lambda: (0, 0)),
        pl.BlockSpec((128, _H1), lambda: (0, 0)),
        pl.BlockSpec((_LM, _H1), lambda: (0, 0)),
        pl.BlockSpec((_LM, _H1), lambda: (0, 0)),
        pl.BlockSpec((1, _H1), lambda: (0, 0)),
        pl.BlockSpec((_H1, _H2), lambda: (0, 0)),
        pl.BlockSpec((1, _H2), lambda: (0, 0)),
        pl.BlockSpec((_H2, 1), lambda: (0, 0)),
        pl.BlockSpec((1, 1), lambda: (0, 0)),
    ],
    out_specs=pl.BlockSpec((_G, 1), lambda: (0, 0)),
    out_shape=jax.ShapeDtypeStruct((_G, 1), _f32),
)


def kernel(x, edge_index, batch, target, drug_lm_embedding,
           protein_lm_embedding, params):
    gin = params["gin"]
    pad = _EP - _E
    pad_idx = jnp.arange(pad, dtype=jnp.int32)
    src = jnp.concatenate(
        [edge_index[0], (pad_idx * 631) % _N]).reshape(_NROWS, _IDXW)
    dst = jnp.concatenate(
        [edge_index[1], _N + pad_idx % (_NPAD - _N)]).reshape(_NROWS, _IDXW)
    zeros = jnp.zeros((_NPAD, _D), _f32)
    batch4 = jnp.concatenate(
        [batch, jnp.full((_NPAD - _N,), _G, jnp.int32)]).reshape(_PPAD, 4)

    r1 = lambda v: v.reshape(1, -1)

    edge_call = _make_edge_call()
    u = _u0_call(x.reshape(_NP4, 4 * _XD), gin[0]["W1"])
    for l in range(4):
        parts = edge_call(u.reshape(_NPAD, _D), src, dst, zeros)
        lyr = gin[l]
        u = _layer_call(u, parts.reshape(_NC, _PPAD, 128), r1(lyr["b1"]),
                        lyr["W2"], r1(lyr["b2"]), r1(lyr["g"]),
                        r1(lyr["be"]), gin[l + 1]["W1"])
    parts = edge_call(u.reshape(_NPAD, _D), src, dst, zeros)
    lyr = gin[4]
    pooled = _layer4_call(u, parts.reshape(_NC, _PPAD, 128), r1(lyr["b1"]),
                          lyr["W2"], r1(lyr["b2"]), r1(lyr["g"]),
                          r1(lyr["be"]), batch4)

    a_mat = params["conv_w"].transpose(2, 0, 1).reshape(_KSZ * _NF, 128)
    emb_t = jnp.pad(params["emb"], ((0, _D - 26), (0, 0))).T
    mt2 = _mt_call(a_mat, emb_t)
    c3 = _conv_call(target.reshape(_G, 1, _SEQ), mt2,
                    params["conv_b"].reshape(_NF, 1))
    c_flat = c3.reshape(_G, _FLAT)

    wxd, bxd = params["fc1_xd"]
    wxt, bxt = params["fc1_xt"]
    w1, b1 = params["fc1"]
    w2, b2 = params["fc2"]
    w3, b3 = params["out"]
    return _head_call(pooled, wxd, r1(bxd), c_flat, wxt, r1(bxt),
                      drug_lm_embedding, protein_lm_embedding,
                      w1[:128], w1[128:256], w1[256:256 + _LM],
                      w1[256 + _LM:], r1(b1), w2, r1(b2), w3, r1(b3))


# speedup vs baseline: 1.0090x; 1.0090x over previous
import functools

import jax
import jax.numpy as jnp
from jax import lax
from jax.experimental import pallas as pl
from jax.experimental.pallas import tpu as pltpu
from jax.experimental.pallas import tpu_sc as plsc

_N = 50000
_E = 800000
_G = 128
_D = 32
_XD = 78
_SEQ = 1000
_KSZ = 8
_NF = 32
_CONV_T = _SEQ - _KSZ + 1

_NC, _NS = 2, 16
_NW = _NC * _NS
_EPW = _E // _NW
_EP = 819200
_IDXW = 128
_NROWS = _EP // _IDXW
_RPW = _NROWS // _NW
_BPW = _RPW // 8
_NPAD = 50048
_ZPW = _NPAD // _NS
_R = 5

_BN = 1.0 / (1.0 + 1e-5) ** 0.5

_f32 = jnp.float32


def _edge_body(u_hbm, src_hbm, dst_hbm, zero_hbm, out_hbm,
               src_v, dst_v, rows_v, acc_sh, isem, gsem, ssem):
    c = lax.axis_index("c")
    s = lax.axis_index("s")
    w = s * _NC + c
    base = w * _RPW

    def idx_load(blk, buf):
        pltpu.async_copy(src_hbm.at[pl.ds(base + blk * 8, 8)],
                         src_v.at[buf], isem)
        pltpu.async_copy(dst_hbm.at[pl.ds(base + blk * 8, 8)],
                         dst_v.at[buf], isem)

    def idx_wait():
        pltpu.make_async_copy(src_hbm.at[pl.ds(0, 8)], src_v.at[0],
                              isem).wait()
        pltpu.make_async_copy(dst_hbm.at[pl.ds(0, 8)], dst_v.at[0],
                              isem).wait()

    def g_issue(buf, row, slot):
        pltpu.async_copy(u_hbm.at[src_v.at[buf, row]], rows_v.at[slot],
                         gsem)

    def g_wait():
        pltpu.make_async_copy(u_hbm.at[src_v.at[0, 0]], rows_v.at[0],
                              gsem).wait()

    def s_issue(buf, row, slot):
        pltpu.sync_copy(rows_v.at[slot], acc_sh.at[dst_v.at[buf, row]],
                        add=True)

    with jax.named_scope("zero_phase"):
        pltpu.sync_copy(zero_hbm.at[pl.ds(s * _ZPW, _ZPW)],
                        acc_sh.at[pl.ds(s * _ZPW, _ZPW)])
        plsc.subcore_barrier()

    def item(B, r, *, first_block=False, last_block=False):
        bb = lax.rem(B, 2)
        nb = lax.rem(B + 1, 2)
        j = B * 8 + r
        if r == 1 and not last_block:
            idx_load(B + 1, nb)
        if r == 3 and not last_block:
            idx_wait()
        if not (last_block and r >= 4):
            if r <= 3:
                g_issue(bb, r + 4, lax.rem(j + 4, _R))
            else:
                g_issue(nb, r - 4, lax.rem(j + 4, _R))
        g_wait()
        s_issue(bb, r, lax.rem(j, _R))

    with jax.named_scope("edge_loop"):
        zero = jnp.zeros((), jnp.int32)
        idx_load(0, 0)
        idx_wait()
        for m in range(4):
            g_issue(0, m, m)
        for r in range(8):
            item(zero, r, first_block=True)

        def blk(B, carry):
            for r in range(8):
                item(B, r)
            return carry

        lax.fori_loop(1, _BPW - 1, blk, 0)

        last = jnp.full((), _BPW - 1, jnp.int32)
        for r in range(8):
            item(last, r, last_block=True)
    with jax.named_scope("export_phase"):
        plsc.subcore_barrier()
        pltpu.sync_copy(acc_sh.at[pl.ds(s * _ZPW, _ZPW)],
                        out_hbm.at[c].at[pl.ds(s * _ZPW, _ZPW)])


@functools.cache
def _make_edge_call():
    return pl.kernel(
        _edge_body,
        out_type=jax.ShapeDtypeStruct((_NC, _NPAD, _D), _f32),
        mesh=plsc.VectorSubcoreMesh(core_axis_name="c", subcore_axis_name="s",
                                    num_cores=_NC, num_subcores=_NS),
        scratch_types=[
            pltpu.VMEM((2, 8, _IDXW), jnp.int32),
            pltpu.VMEM((2, 8, _IDXW), jnp.int32),
            pltpu.VMEM((_R, _IDXW, _D), _f32),
            pltpu.VMEM_SHARED((_NPAD, _D), _f32),
            pltpu.SemaphoreType.DMA,
            pltpu.SemaphoreType.DMA,
            pltpu.SemaphoreType.DMA,
        ],
        compiler_params=pltpu.CompilerParams(use_tc_tiling_on_sc=False),
    )


_NP4 = _N // 4
_PPAD = _NPAD // 4
_RB = 3128
_NRB = _PPAD // _RB


def _bd4(w):
    z = jnp.zeros(w.shape, w.dtype)
    rows = [jnp.concatenate([z] * k + [w] + [z] * (3 - k), axis=1)
            for k in range(4)]
    return jnp.concatenate(rows, axis=0)


def _t4(v):
    return jnp.concatenate([v, v, v, v], axis=1)


def _u0_body(x_ref, w_ref, o_ref):
    u = jnp.dot(x_ref[...], _bd4(w_ref[...]), preferred_element_type=_f32)
    o_ref[...] = jnp.concatenate(
        [u, jnp.zeros((_PPAD - _NP4, 128), _f32)], axis=0)


_u0_call = pl.pallas_call(
    _u0_body,
    in_specs=[
        pl.BlockSpec((_NP4, 4 * _XD), lambda: (0, 0)),
        pl.BlockSpec((_XD, _D), lambda: (0, 0)),
    ],
    out_specs=pl.BlockSpec((_PPAD, 128), lambda: (0, 0)),
    out_shape=jax.ShapeDtypeStruct((_PPAD, 128), _f32),
)


def _mlp(u_ref, p_ref, b1_ref, w2_ref, b2_ref, g_ref, be_ref):
    z = jnp.maximum(u_ref[...] + p_ref[0] + p_ref[1] + _t4(b1_ref[...]), 0.0)
    z = jnp.maximum(jnp.dot(z, _bd4(w2_ref[...]),
                            preferred_element_type=_f32)
                    + _t4(b2_ref[...]), 0.0)
    return z * (_t4(g_ref[...]) * _BN) + _t4(be_ref[...])


def _layer_body(u_ref, p_ref, b1_ref, w2_ref, b2_ref, g_ref, be_ref,
                w1n_ref, o_ref):
    h = _mlp(u_ref, p_ref, b1_ref, w2_ref, b2_ref, g_ref, be_ref)
    o_ref[...] = jnp.dot(h, _bd4(w1n_ref[...]),
                         preferred_element_type=_f32)


_layer_call = pl.pallas_call(
    _layer_body,
    grid=(_NRB,),
    in_specs=[
        pl.BlockSpec((_RB, 128), lambda i: (i, 0)),
        pl.BlockSpec((_NC, _RB, 128), lambda i: (0, i, 0)),
        pl.BlockSpec((1, _D), lambda i: (0, 0)),
        pl.BlockSpec((_D, _D), lambda i: (0, 0)),
        pl.BlockSpec((1, _D), lambda i: (0, 0)),
        pl.BlockSpec((1, _D), lambda i: (0, 0)),
        pl.BlockSpec((1, _D), lambda i: (0, 0)),
        pl.BlockSpec((_D, _D), lambda i: (0, 0)),
    ],
    out_specs=pl.BlockSpec((_RB, 128), lambda i: (i, 0)),
    out_shape=jax.ShapeDtypeStruct((_PPAD, 128), _f32),
)


def _layer4_body(u_ref, p_ref, b1_ref, w2_ref, b2_ref, g_ref, be_ref,
                 bt_ref, o_ref):
    h = _mlp(u_ref, p_ref, b1_ref, w2_ref, b2_ref, g_ref, be_ref)
    iota = lax.broadcasted_iota(jnp.int32, (1, _G), 1)
    part = jnp.zeros((_G, _D), _f32)
    for k in range(4):
        onek = (bt_ref[:, k:k + 1] == iota).astype(_f32)
        part = part + lax.dot_general(
            onek, h[:, 32 * k:32 * k + 32], (((0,), (0,)), ((), ())),
            preferred_element_type=_f32)
    i = pl.program_id(0)

    @pl.when(i == 0)
    def _init():
        o_ref[...] = part

    @pl.when(i > 0)
    def _acc():
        o_ref[...] += part


_layer4_call = pl.pallas_call(
    _layer4_body,
    grid=(_NRB,),
    in_specs=[
        pl.BlockSpec((_RB, 128), lambda i: (i, 0)),
        pl.BlockSpec((_NC, _RB, 128), lambda i: (0, i, 0)),
        pl.BlockSpec((1, _D), lambda i: (0, 0)),
        pl.BlockSpec((_D, _D), lambda i: (0, 0)),
        pl.BlockSpec((1, _D), lambda i: (0, 0)),
        pl.BlockSpec((1, _D), lambda i: (0, 0)),
        pl.BlockSpec((1, _D), lambda i: (0, 0)),
        pl.BlockSpec((_RB, 4), lambda i: (i, 0)),
    ],
    out_specs=pl.BlockSpec((_G, _D), lambda i: (0, 0)),
    out_shape=jax.ShapeDtypeStruct((_G, _D), _f32),
)


def _mt_body(a_ref, e_ref, o_ref):
    o_ref[...] = jnp.dot(a_ref[...], e_ref[...],
                         preferred_element_type=_f32)


_mt_call = pl.pallas_call(
    _mt_body,
    in_specs=[
        pl.BlockSpec((_KSZ * _NF, 128), lambda: (0, 0)),
        pl.BlockSpec((128, _D), lambda: (0, 0)),
    ],
    out_specs=pl.BlockSpec((_KSZ * _NF, _D), lambda: (0, 0)),
    out_shape=jax.ShapeDtypeStruct((_KSZ * _NF, _D), _f32),
)

_GB = 8


def _conv_body(t_ref, m_ref, cb_ref, o_ref):
    iota_v = lax.broadcasted_iota(jnp.int32, (_D, 1), 0)
    for g in range(_GB):
        tgt = t_ref[g]
        onehot = (tgt == iota_v).astype(_f32)
        acc = jnp.zeros((_NF, _CONV_T), _f32)
        for k in range(_KSZ):
            mt_k = m_ref[pl.ds(k * _NF, _NF), :]
            p = jnp.dot(mt_k, onehot, preferred_element_type=_f32)
            acc = acc + p[:, k:k + _CONV_T]
        o_ref[g] = jnp.maximum(acc + cb_ref[...], 0.0)


_conv_call = pl.pallas_call(
    _conv_body,
    grid=(_G // _GB,),
    in_specs=[
        pl.BlockSpec((_GB, 1, _SEQ), lambda i: (i, 0, 0)),
        pl.BlockSpec((_KSZ * _NF, _D), lambda i: (0, 0)),
        pl.BlockSpec((_NF, 1), lambda i: (0, 0)),
    ],
    out_specs=pl.BlockSpec((_GB, _NF, _CONV_T), lambda i: (i, 0, 0)),
    out_shape=jax.ShapeDtypeStruct((_G, _NF, _CONV_T), _f32),
)

_FLAT = _NF * _CONV_T
_LM = 1024
_H1 = 1024
_H2 = 256


def _head_body(pooled_ref, wxd_ref, bxd_ref, c_ref, wxt_ref, bxt_ref,
               drug_ref, prot_ref, w1a_ref, w1b_ref, w1c_ref, w1d_ref,
               b1_ref, w2_ref, b2_ref, w3_ref, b3_ref, o_ref):
    xd = jnp.maximum(jnp.dot(pooled_ref[...], wxd_ref[...],
                             preferred_element_type=_f32) + bxd_ref[...], 0.0)
    xt = jnp.maximum(jnp.dot(c_ref[...], wxt_ref[...],
                             preferred_element_type=_f32) + bxt_ref[...], 0.0)
    y = (jnp.dot(xd, w1a_ref[...], preferred_element_type=_f32)
         + jnp.dot(xt, w1b_ref[...], preferred_element_type=_f32)
         + jnp.dot(drug_ref[...], w1c_ref[...], preferred_element_type=_f32)
         + jnp.dot(prot_ref[...], w1d_ref[...], preferred_element_type=_f32)
         + b1_ref[...])
    y = jnp.maximum(y, 0.0)
    y = jnp.maximum(jnp.dot(y, w2_ref[...], preferred_element_type=_f32)
                    + b2_ref[...], 0.0)
    o_ref[...] = jnp.dot(y, w3_ref[...],
                         preferred_element_type=_f32) + b3_ref[...]


_head_call = pl.pallas_call(
    _head_body,
    in_specs=[
        pl.BlockSpec((_G, _D), lambda: (0, 0)),
        pl.BlockSpec((_D, 128), lambda: (0, 0)),
        pl.BlockSpec((1, 128), lambda: (0, 0)),
        pl.BlockSpec((_G, _FLAT), lambda: (0, 0)),
        pl.BlockSpec((_FLAT, 128), lambda: (0, 0)),
        pl.BlockSpec((1, 128), lambda: (0, 0)),
        pl.BlockSpec((_G, _LM), lambda: (0, 0)),
        pl.BlockSpec((_G, _LM), lambda: (0, 0)),
        pl.BlockSpec((128, _H1), lambda: (0, 0)),
        pl.BlockSpec((128, _H1), lambda: (0, 0)),
        pl.BlockSpec((_LM, _H1), lambda: (0, 0)),
        pl.BlockSpec((_LM, _H1), lambda: (0, 0)),
        pl.BlockSpec((1, _H1), lambda: (0, 0)),
        pl.BlockSpec((_H1, _H2), lambda: (0, 0)),
        pl.BlockSpec((1, _H2), lambda: (0, 0)),
        pl.BlockSpec((_H2, 1), lambda: (0, 0)),
        pl.BlockSpec((1, 1), lambda: (0, 0)),
    ],
    out_specs=pl.BlockSpec((_G, 1), lambda: (0, 0)),
    out_shape=jax.ShapeDtypeStruct((_G, 1), _f32),
)


def kernel(x, edge_index, batch, target, drug_lm_embedding,
           protein_lm_embedding, params):
    gin = params["gin"]
    pad = _EP - _E
    pad_idx = jnp.arange(pad, dtype=jnp.int32)
    src = jnp.concatenate(
        [edge_index[0], (pad_idx * 631) % _N]).reshape(_NROWS, _IDXW)
    dst = jnp.concatenate(
        [edge_index[1], _N + pad_idx % (_NPAD - _N)]).reshape(_NROWS, _IDXW)
    zeros = jnp.zeros((_NPAD, _D), _f32)
    batch4 = jnp.concatenate(
        [batch, jnp.full((_NPAD - _N,), _G, jnp.int32)]).reshape(_PPAD, 4)

    r1 = lambda v: v.reshape(1, -1)

    edge_call = _make_edge_call()
    u = _u0_call(x.reshape(_NP4, 4 * _XD), gin[0]["W1"])
    for l in range(4):
        parts = edge_call(u.reshape(_NPAD, _D), src, dst, zeros)
        lyr = gin[l]
        u = _layer_call(u, parts.reshape(_NC, _PPAD, 128), r1(lyr["b1"]),
                        lyr["W2"], r1(lyr["b2"]), r1(lyr["g"]),
                        r1(lyr["be"]), gin[l + 1]["W1"])
    parts = edge_call(u.reshape(_NPAD, _D), src, dst, zeros)
    lyr = gin[4]
    pooled = _layer4_call(u, parts.reshape(_NC, _PPAD, 128), r1(lyr["b1"]),
                          lyr["W2"], r1(lyr["b2"]), r1(lyr["g"]),
                          r1(lyr["be"]), batch4)

    a_mat = params["conv_w"].transpose(2, 0, 1).reshape(_KSZ * _NF, 128)
    emb_t = jnp.pad(params["emb"], ((0, _D - 26), (0, 0))).T
    mt2 = _mt_call(a_mat, emb_t)
    c3 = _conv_call(target.reshape(_G, 1, _SEQ), mt2,
                    params["conv_b"].reshape(_NF, 1))
    c_flat = c3.reshape(_G, _FLAT)

    wxd, bxd = params["fc1_xd"]
    wxt, bxt = params["fc1_xt"]
    w1, b1 = params["fc1"]
    w2, b2 = params["fc2"]
    w3, b3 = params["out"]
    return _head_call(pooled, wxd, r1(bxd), c_flat, wxt, r1(bxt),
                      drug_lm_embedding, protein_lm_embedding,
                      w1[:128], w1[128:256], w1[256:256 + _LM],
                      w1[256 + _LM:], r1(b1), w2, r1(b2), w3, r1(b3))
